# baseline (speedup 1.0000x reference)
import jax
import jax.numpy as jnp
from jax import lax
from jax.experimental import pallas as pl
from jax.experimental.pallas import tpu as pltpu
from jax.experimental.pallas import tpu_sc as plsc

E = 16
H = 768
F = 2048
T = 64
NCH = 4
FC = F // NCH
NBUF = 6
NTILES = (E + 1) * NCH

NC = 2
NS = 16
TPW = T // (NC * NS)
NK = H // 16


def _sc_gating_body(x_hbm, gw_hbm, comb_hbm, gw_v, xrow_v, comb_v):
    wid = lax.axis_index("s") * NC + lax.axis_index("c")
    base = wid * TPW
    pltpu.sync_copy(gw_hbm, gw_v)
    pltpu.sync_copy(x_hbm.at[pl.ds(base, TPW)], xrow_v)
    iota = lax.iota(jnp.int32, 16)
    perms = [jnp.bitwise_xor(iota, k) for k in (8, 4, 2, 1)]

    def shuf(v, p):
        return v.at[p].get(mode="promise_in_bounds")

    def bred(v, op):
        for p in perms:
            v = op(v, shuf(v, p))
        return v

    def bf16_round(v):
        t = v * 65537.0
        return t - (t - v)

    for tt in range(TPW):
        def ebody(e, lvec):
            def kbody(k, acc):
                return acc + (bf16_round(xrow_v[tt, pl.ds(k * 16, 16)]) *
                              bf16_round(gw_v[e, pl.ds(k * 16, 16)]))
            acc = lax.fori_loop(0, NK, kbody, jnp.zeros((16,), jnp.float32))
            return jnp.where(iota == e, bred(acc, jnp.add), lvec)

        lvec = lax.fori_loop(0, E, ebody, jnp.zeros((16,), jnp.float32))
        m = bred(lvec, jnp.maximum)
        ex = jnp.exp(lvec - m)
        sc = ex / bred(ex, jnp.add)
        m1 = bred(sc, jnp.maximum)
        i1 = bred(jnp.where(sc == m1, iota, E), jnp.minimum)
        masked = jnp.where(iota == i1, -1e30, sc)
        m2 = bred(masked, jnp.maximum)
        i2 = bred(jnp.where(masked == m2, iota, E), jnp.minimum)
        denom = m1 + m2 + 1e-20
        comb_v[tt, :] = (jnp.where(iota == i1, m1 / denom, 0.0) +
                         jnp.where(iota == i2, m2 / denom, 0.0))

    pltpu.sync_copy(comb_v, comb_hbm.at[pl.ds(base, TPW)])


def _sc_gating(x2, gate_w):
    return pl.kernel(
        _sc_gating_body,
        out_type=jax.ShapeDtypeStruct((T, E), jnp.float32),
        mesh=plsc.VectorSubcoreMesh(core_axis_name="c", subcore_axis_name="s"),
        scratch_types=[
            pltpu.VMEM((E, H), jnp.float32),
            pltpu.VMEM((TPW, H), jnp.float32),
            pltpu.VMEM((TPW, E), jnp.float32),
        ],
    )(x2, gate_w)


def _ffn_kernel(x_ref, comb_ref, wg_hbm, wu_hbm, wd_hbm, swg_hbm, swu_hbm,
                swd_hbm, out_ref, wg_buf, wu_buf, wd_buf, sem):

    def issue(t, slot):
        e = t // NCH
        f0 = (t % NCH) * FC

        @pl.when(e < E)
        def _():
            pltpu.make_async_copy(
                wg_hbm.at[e, :, pl.ds(f0, FC)], wg_buf.at[slot],
                sem.at[0, slot]).start()
            pltpu.make_async_copy(
                wu_hbm.at[e, :, pl.ds(f0, FC)], wu_buf.at[slot],
                sem.at[1, slot]).start()
            pltpu.make_async_copy(
                wd_hbm.at[e, pl.ds(f0, FC), :], wd_buf.at[slot],
                sem.at[2, slot]).start()

        @pl.when(e == E)
        def _():
            pltpu.make_async_copy(
                swg_hbm.at[:, pl.ds(f0, FC)], wg_buf.at[slot],
                sem.at[0, slot]).start()
            pltpu.make_async_copy(
                swu_hbm.at[:, pl.ds(f0, FC)], wu_buf.at[slot],
                sem.at[1, slot]).start()
            pltpu.make_async_copy(
                swd_hbm.at[pl.ds(f0, FC), :], wd_buf.at[slot],
                sem.at[2, slot]).start()

    def wait_one(kind, hbm, buf, slot):
        pltpu.make_async_copy(
            hbm.at[0, :, pl.ds(0, FC)] if kind < 2
            else hbm.at[0, pl.ds(0, FC), :],
            buf.at[slot], sem.at[kind, slot]).wait()

    for t in range(NBUF - 1):
        issue(jnp.int32(t), jnp.int32(t))

    out_ref[...] = jnp.zeros_like(out_ref)

    def body(t, _):
        slot = jax.lax.rem(t, NBUF)
        nxt = t + NBUF - 1

        e = t // NCH
        x = x_ref[...].astype(jnp.bfloat16)
        wait_one(0, wg_hbm, wg_buf, slot)
        g = jnp.dot(x, wg_buf[slot].astype(jnp.bfloat16),
                    preferred_element_type=jnp.float32)
        wait_one(1, wu_hbm, wu_buf, slot)
        u = jnp.dot(x, wu_buf[slot].astype(jnp.bfloat16),
                    preferred_element_type=jnp.float32)
        act = g * jax.lax.logistic(g) * u
        wait_one(2, wd_hbm, wd_buf, slot)

        @pl.when(nxt < NTILES)
        def _():
            issue(nxt, jax.lax.rem(nxt, NBUF))

        o = jnp.dot(act.astype(jnp.bfloat16), wd_buf[slot].astype(jnp.bfloat16),
                    preferred_element_type=jnp.float32)
        lane = jax.lax.broadcasted_iota(jnp.int32, (T, E), 1)
        w_col = jnp.sum(jnp.where(lane == e, comb_ref[...], 0.0),
                        axis=-1, keepdims=True)
        w_col = w_col + jnp.where(e == E, 1.0, 0.0)
        out_ref[...] += w_col * o
        return 0

    jax.lax.fori_loop(0, NTILES, body, 0)


@jax.jit
def kernel(x, gate_w, Wg, Wu, Wd, SWg, SWu, SWd):
    b, s, h = x.shape
    x2 = x.reshape(-1, h)

    comb = _sc_gating(x2, gate_w)

    out = pl.pallas_call(
        _ffn_kernel,
        in_specs=[
            pl.BlockSpec(memory_space=pltpu.MemorySpace.VMEM),
            pl.BlockSpec(memory_space=pltpu.MemorySpace.VMEM),
            pl.BlockSpec(memory_space=pltpu.MemorySpace.HBM),
            pl.BlockSpec(memory_space=pltpu.MemorySpace.HBM),
            pl.BlockSpec(memory_space=pltpu.MemorySpace.HBM),
            pl.BlockSpec(memory_space=pltpu.MemorySpace.HBM),
            pl.BlockSpec(memory_space=pltpu.MemorySpace.HBM),
            pl.BlockSpec(memory_space=pltpu.MemorySpace.HBM),
        ],
        out_specs=pl.BlockSpec(memory_space=pltpu.MemorySpace.VMEM),
        out_shape=jax.ShapeDtypeStruct((T, H), jnp.float32),
        scratch_shapes=[
            pltpu.VMEM((NBUF, H, FC), jnp.float32),
            pltpu.VMEM((NBUF, H, FC), jnp.float32),
            pltpu.VMEM((NBUF, FC, H), jnp.float32),
            pltpu.SemaphoreType.DMA((3, NBUF)),
        ],
    )(x2, comb, Wg, Wu, Wd, SWg, SWu, SWd)

    return out.reshape(b, s, h)


# speedup vs baseline: 1.2490x; 1.2490x over previous
import jax
import jax.numpy as jnp
from jax.experimental import pallas as pl
from jax.experimental.pallas import tpu as pltpu

E = 16
H = 768
F = 2048
T = 64
NCH = 4
FC = F // NCH
NBUF = 6
NTILES = (E + 1) * NCH


def _gating(x, gw):
    logits = jax.lax.dot_general(
        x.astype(jnp.bfloat16), gw.astype(jnp.bfloat16),
        (((1,), (1,)), ((), ())),
        preferred_element_type=jnp.float32)
    m = jnp.max(logits, axis=-1, keepdims=True)
    ex = jnp.exp(logits - m)
    scores = ex / jnp.sum(ex, axis=-1, keepdims=True)
    iota = jax.lax.broadcasted_iota(jnp.int32, (T, E), 1)
    m1 = jnp.max(scores, axis=-1, keepdims=True)
    i1 = jnp.min(jnp.where(scores == m1, iota, E), axis=-1, keepdims=True)
    masked = jnp.where(iota == i1, -jnp.inf, scores)
    m2 = jnp.max(masked, axis=-1, keepdims=True)
    i2 = jnp.min(jnp.where(masked == m2, iota, E), axis=-1, keepdims=True)
    denom = m1 + m2 + 1e-20
    comb = jnp.where(iota == i1, m1 / denom, 0.0)
    return comb + jnp.where(iota == i2, m2 / denom, 0.0)


def _ffn_kernel(x_ref, gw_ref, wg_hbm, wu_hbm, wd_hbm, swg_hbm, swu_hbm,
                swd_hbm, out_ref, wg_buf, wu_buf, wd_buf, comb_ref, sem):

    def issue(t, slot):
        e = t // NCH
        f0 = (t % NCH) * FC

        @pl.when(e < E)
        def _():
            pltpu.make_async_copy(
                wg_hbm.at[e, :, pl.ds(f0, FC)], wg_buf.at[slot],
                sem.at[0, slot]).start()
            pltpu.make_async_copy(
                wu_hbm.at[e, :, pl.ds(f0, FC)], wu_buf.at[slot],
                sem.at[1, slot]).start()
            pltpu.make_async_copy(
                wd_hbm.at[e, pl.ds(f0, FC), :], wd_buf.at[slot],
                sem.at[2, slot]).start()

        @pl.when(e == E)
        def _():
            pltpu.make_async_copy(
                swg_hbm.at[:, pl.ds(f0, FC)], wg_buf.at[slot],
                sem.at[0, slot]).start()
            pltpu.make_async_copy(
                swu_hbm.at[:, pl.ds(f0, FC)], wu_buf.at[slot],
                sem.at[1, slot]).start()
            pltpu.make_async_copy(
                swd_hbm.at[pl.ds(f0, FC), :], wd_buf.at[slot],
                sem.at[2, slot]).start()

    def wait_one(kind, hbm, buf, slot):
        pltpu.make_async_copy(
            hbm.at[0, :, pl.ds(0, FC)] if kind < 2
            else hbm.at[0, pl.ds(0, FC), :],
            buf.at[slot], sem.at[kind, slot]).wait()

    for t in range(NBUF - 1):
        issue(jnp.int32(t), jnp.int32(t))

    comb_ref[...] = _gating(x_ref[...], gw_ref[...])
    out_ref[...] = jnp.zeros_like(out_ref)

    def body(t, _):
        slot = jax.lax.rem(t, NBUF)
        nxt = t + NBUF - 1

        e = t // NCH
        x = x_ref[...].astype(jnp.bfloat16)
        wait_one(0, wg_hbm, wg_buf, slot)
        g = jnp.dot(x, wg_buf[slot].astype(jnp.bfloat16),
                    preferred_element_type=jnp.float32)
        wait_one(1, wu_hbm, wu_buf, slot)
        u = jnp.dot(x, wu_buf[slot].astype(jnp.bfloat16),
                    preferred_element_type=jnp.float32)
        act = g * jax.lax.logistic(g) * u
        wait_one(2, wd_hbm, wd_buf, slot)

        @pl.when(nxt < NTILES)
        def _():
            issue(nxt, jax.lax.rem(nxt, NBUF))

        o = jnp.dot(act.astype(jnp.bfloat16), wd_buf[slot].astype(jnp.bfloat16),
                    preferred_element_type=jnp.float32)
        lane = jax.lax.broadcasted_iota(jnp.int32, (T, E), 1)
        w_col = jnp.sum(jnp.where(lane == e, comb_ref[...], 0.0),
                        axis=-1, keepdims=True)
        w_col = w_col + jnp.where(e == E, 1.0, 0.0)
        out_ref[...] += w_col * o
        return 0

    jax.lax.fori_loop(0, NTILES, body, 0)


@jax.jit
def kernel(x, gate_w, Wg, Wu, Wd, SWg, SWu, SWd):
    b, s, h = x.shape
    x2 = x.reshape(-1, h)

    out = pl.pallas_call(
        _ffn_kernel,
        in_specs=[
            pl.BlockSpec(memory_space=pltpu.MemorySpace.VMEM),
            pl.BlockSpec(memory_space=pltpu.MemorySpace.VMEM),
            pl.BlockSpec(memory_space=pltpu.MemorySpace.HBM),
            pl.BlockSpec(memory_space=pltpu.MemorySpace.HBM),
            pl.BlockSpec(memory_space=pltpu.MemorySpace.HBM),
            pl.BlockSpec(memory_space=pltpu.MemorySpace.HBM),
            pl.BlockSpec(memory_space=pltpu.MemorySpace.HBM),
            pl.BlockSpec(memory_space=pltpu.MemorySpace.HBM),
        ],
        out_specs=pl.BlockSpec(memory_space=pltpu.MemorySpace.VMEM),
        out_shape=jax.ShapeDtypeStruct((T, H), jnp.float32),
        scratch_shapes=[
            pltpu.VMEM((NBUF, H, FC), jnp.float32),
            pltpu.VMEM((NBUF, H, FC), jnp.float32),
            pltpu.VMEM((NBUF, FC, H), jnp.float32),
            pltpu.VMEM((T, E), jnp.float32),
            pltpu.SemaphoreType.DMA((3, NBUF)),
        ],
    )(x2, gate_w, Wg, Wu, Wd, SWg, SWu, SWd)

    return out.reshape(b, s, h)
